# baseline (speedup 1.0000x reference)
import functools

import jax
import jax.numpy as jnp
from jax import lax
from jax.experimental import pallas as pl
from jax.experimental.pallas import tpu as pltpu
from jax.experimental.pallas import tpu_sc as plsc

N_NODES = 10000
N_EDGES = 320000
D_FEAT = 128
N_HIDDEN = 128
N_CLASSES = 64

NC = 2
NS = 16
L = 16
NW = NC * NS

N_PAD = 10240
EDGE_K = 80

_MESH = dict(core_axis_name="c", subcore_axis_name="s")


def _newton_rsqrt(d):
    i = lax.bitcast_convert_type(d, jnp.int32)
    i = jnp.int32(0x5F3759DF) - lax.shift_right_logical(i, 1)
    y = lax.bitcast_convert_type(i, jnp.float32)
    for _ in range(3):
        y = y * (1.5 - 0.5 * d * y * y)
    return y


EPT_DEG = N_EDGES // NS
NODES_PER_TILE = N_PAD // NS


@functools.partial(
    pl.kernel,
    out_type=[
        jax.ShapeDtypeStruct((N_PAD,), jnp.float32),
        jax.ShapeDtypeStruct((N_PAD,), jnp.float32),
    ],
    mesh=plsc.VectorSubcoreMesh(**_MESH),
    compiler_params=pltpu.CompilerParams(
        needs_layout_passes=False, use_tc_tiling_on_sc=False
    ),
    scratch_types=[
        pltpu.VMEM((EPT_DEG,), jnp.int32),
        pltpu.VMEM((N_PAD,), jnp.float32),
        pltpu.VMEM((NS, NODES_PER_TILE), jnp.float32),
        pltpu.VMEM_SHARED((NS, N_PAD), jnp.float32),
    ],
)
def _sc_norms(adj_hbm, nsrc_hbm, ndst_hbm, idx_v, hist_v, stripe_v, parts_sh):
    c = lax.axis_index("c")
    s = lax.axis_index("s")

    pltpu.sync_copy(adj_hbm.at[c, pl.ds(s * EPT_DEG, EPT_DEG)], idx_v)

    zeros = jnp.zeros((L,), jnp.float32)

    def _zero(j, _):
        hist_v[pl.ds(j * L, L)] = zeros
        return 0

    lax.fori_loop(0, N_PAD // L, _zero, 0)

    ones = jnp.ones((L,), jnp.float32)

    def _count(i, _):
        idx = idx_v[pl.ds(i * L, L)]
        plsc.addupdate_scatter(hist_v, [idx], ones)
        return 0

    lax.fori_loop(0, EPT_DEG // L, _count, 0)

    pltpu.sync_copy(hist_v, parts_sh.at[s])
    plsc.subcore_barrier()
    base = s * NODES_PER_TILE
    pltpu.sync_copy(parts_sh.at[:, pl.ds(base, NODES_PER_TILE)], stripe_v)

    def _reduce(j, _):
        deg = jnp.zeros((L,), jnp.float32)
        for p in range(NS):
            deg = deg + stripe_v[p, pl.ds(j * L, L)]
        norm = jnp.where(deg > 0.0, _newton_rsqrt(jnp.maximum(deg, 1.0)), 0.0)
        hist_v[pl.ds(j * L, L)] = norm
        return 0

    lax.fori_loop(0, NODES_PER_TILE // L, _reduce, 0)

    @pl.when(c == 0)
    def _():
        pltpu.sync_copy(
            hist_v.at[pl.ds(0, NODES_PER_TILE)],
            nsrc_hbm.at[pl.ds(base, NODES_PER_TILE)],
        )

    @pl.when(c == 1)
    def _():
        pltpu.sync_copy(
            hist_v.at[pl.ds(0, NODES_PER_TILE)],
            ndst_hbm.at[pl.ds(base, NODES_PER_TILE)],
        )


EPT_AGG = N_EDGES // NW


AGG_K = 128
N_FULL = EPT_AGG // AGG_K
TAIL = EPT_AGG - N_FULL * AGG_K


def _make_sc_agg(feat):
    rows_per_tile = N_PAD // NS

    @functools.partial(
        pl.kernel,
        out_type=jax.ShapeDtypeStruct((NC, N_PAD, feat), jnp.float32),
        mesh=plsc.VectorSubcoreMesh(**_MESH),
        compiler_params=pltpu.CompilerParams(use_tc_tiling_on_sc=False),
        scratch_types=[
            pltpu.VMEM((EPT_AGG,), jnp.int32),
            [pltpu.VMEM((AGG_K,), jnp.int32)] * 2,
            pltpu.VMEM((TAIL,), jnp.int32),
            [pltpu.VMEM((AGG_K, feat), jnp.float32)] * 2,
            pltpu.VMEM_SHARED((N_PAD, feat), jnp.float32),
            [pltpu.SemaphoreType.DMA] * 2,
            [pltpu.SemaphoreType.DMA] * 2,
            [pltpu.SemaphoreType.DMA] * 2,
        ],
    )
    def _sc_agg(h_hbm, adj_hbm, out_hbm, src_v, didx, didx_t, rows,
                acc_sh, gsem, dsem, ssem):
        c = lax.axis_index("c")
        s = lax.axis_index("s")
        wid = s * NC + c
        base = wid * EPT_AGG

        pltpu.sync_copy(adj_hbm.at[0, pl.ds(base, EPT_AGG)], src_v)

        zeros = jnp.zeros((L,), jnp.float32)

        def _zrow(i, _):
            for t in range(feat // L):
                rows[0][i, pl.ds(t * L, L)] = zeros
            return 0

        lax.fori_loop(0, AGG_K, _zrow, 0)
        for t in range(rows_per_tile // AGG_K):
            pltpu.sync_copy(
                rows[0], acc_sh.at[pl.ds(s * rows_per_tile + t * AGG_K, AGG_K)]
            )
        plsc.subcore_barrier()

        def _gather(ci, b):
            pltpu.async_copy(
                h_hbm.at[src_v.at[pl.ds(ci * AGG_K, AGG_K)]], rows[b], gsem[b]
            )
            pltpu.async_copy(
                adj_hbm.at[1, pl.ds(base + ci * AGG_K, AGG_K)], didx[b], dsem[b]
            )

        def _wait_g(b):
            pltpu.make_async_copy(h_hbm.at[pl.ds(0, AGG_K)], rows[b], gsem[b]).wait()
            pltpu.make_async_copy(
                adj_hbm.at[1, pl.ds(base, AGG_K)], didx[b], dsem[b]
            ).wait()

        def _scat(b):
            pltpu.async_copy(rows[b], acc_sh.at[didx[b]], ssem[b], add=True)

        def _wait_s(b):
            pltpu.make_async_copy(h_hbm.at[pl.ds(0, AGG_K)], rows[b], ssem[b]).wait()

        _gather(0, 0)
        _wait_g(0)
        _scat(0)
        _gather(1, 1)
        _wait_g(1)
        _wait_s(0)
        _scat(1)
        _gather(2, 0)

        def _body(i, _):
            c0 = 2 * i
            _wait_g(0)
            _wait_s(1)
            _scat(0)
            _gather(c0 + 1, 1)
            _wait_g(1)
            _wait_s(0)
            _scat(1)

            @pl.when(c0 + 2 < N_FULL)
            def _():
                _gather(c0 + 2, 0)

            return 0

        lax.fori_loop(1, N_FULL // 2, _body, 0)
        _wait_s(1)

        toff = base + N_FULL * AGG_K
        pltpu.sync_copy(adj_hbm.at[1, pl.ds(toff, TAIL)], didx_t)
        pltpu.async_copy(
            h_hbm.at[src_v.at[pl.ds(N_FULL * AGG_K, TAIL)]],
            rows[0].at[pl.ds(0, TAIL)],
            gsem[0],
        )
        pltpu.make_async_copy(
            h_hbm.at[pl.ds(0, TAIL)], rows[0].at[pl.ds(0, TAIL)], gsem[0]
        ).wait()
        pltpu.sync_copy(rows[0].at[pl.ds(0, TAIL)], acc_sh.at[didx_t], add=True)

        plsc.subcore_barrier()
        sl = pl.ds(s * rows_per_tile, rows_per_tile)
        pltpu.sync_copy(acc_sh.at[sl], out_hbm.at[c, sl])

    return _sc_agg


_sc_agg_h = _make_sc_agg(N_HIDDEN)
_sc_agg_c = _make_sc_agg(N_CLASSES)


BM = 512
GRID = N_PAD // BM


def _tc_mm1_body(x_ref, w_ref, o_ref):
    o_ref[...] = jnp.dot(x_ref[...], w_ref[...], preferred_element_type=jnp.float32)


def _tc_scale_body(y_ref, ns_ref, o_ref):
    ns = ns_ref[...]
    o_ref[...] = jnp.where(ns > 0.0, y_ref[...] * ns, 0.0)


def _tc_l2_body(p_ref, nd_ref, ns_ref, b_ref, w_ref, o_ref):
    t = (p_ref[0] + p_ref[1]) * nd_ref[...] + b_ref[...]
    t = jnp.maximum(t, 0.0) * ns_ref[...]
    o_ref[...] = jnp.dot(t, w_ref[...], preferred_element_type=jnp.float32)


def _tc_out_body(q_ref, nd_ref, b_ref, o_ref):
    z = (q_ref[0] + q_ref[1]) * nd_ref[...] + b_ref[...]
    m = jnp.max(z, axis=1, keepdims=True)
    e = jnp.exp(z - m)
    o_ref[...] = z - m - jnp.log(jnp.sum(e, axis=1, keepdims=True))


def _row_spec(cols):
    return pl.BlockSpec((BM, cols), lambda i: (i, 0))


def _full_spec(rows, cols):
    return pl.BlockSpec((rows, cols), lambda i: (0, 0))


def _pair_spec(cols):
    return pl.BlockSpec((2, BM, cols), lambda i: (0, i, 0))


def kernel(features, adj_metrix, W1, b1, W2, b2):
    y = pl.pallas_call(
        _tc_mm1_body,
        grid=(GRID,),
        in_specs=[_row_spec(D_FEAT), _full_spec(D_FEAT, N_HIDDEN)],
        out_specs=_row_spec(N_HIDDEN),
        out_shape=jax.ShapeDtypeStruct((N_PAD, N_HIDDEN), jnp.float32),
    )(features, W1)

    nsrc_flat, ndst_flat = _sc_norms(adj_metrix)
    nsrc = nsrc_flat.reshape(N_PAD, 1)
    ndst = ndst_flat.reshape(N_PAD, 1)

    h0 = pl.pallas_call(
        _tc_scale_body,
        grid=(GRID,),
        in_specs=[_row_spec(N_HIDDEN), _row_spec(1)],
        out_specs=_row_spec(N_HIDDEN),
        out_shape=jax.ShapeDtypeStruct((N_PAD, N_HIDDEN), jnp.float32),
    )(y, nsrc)

    p = _sc_agg_h(h0, adj_metrix)

    h1 = pl.pallas_call(
        _tc_l2_body,
        grid=(GRID,),
        in_specs=[
            _pair_spec(N_HIDDEN),
            _row_spec(1),
            _row_spec(1),
            _full_spec(1, N_HIDDEN),
            _full_spec(N_HIDDEN, N_CLASSES),
        ],
        out_specs=_row_spec(N_CLASSES),
        out_shape=jax.ShapeDtypeStruct((N_PAD, N_CLASSES), jnp.float32),
    )(p, ndst, nsrc, b1.reshape(1, N_HIDDEN), W2)

    q = _sc_agg_c(h1, adj_metrix)

    out = pl.pallas_call(
        _tc_out_body,
        grid=(GRID,),
        in_specs=[
            _pair_spec(N_CLASSES),
            _row_spec(1),
            _full_spec(1, N_CLASSES),
        ],
        out_specs=_row_spec(N_CLASSES),
        out_shape=jax.ShapeDtypeStruct((N_NODES, N_CLASSES), jnp.float32),
    )(q, ndst, b2.reshape(1, N_CLASSES))

    return out


# speedup vs baseline: 1.1547x; 1.1547x over previous
import functools

import jax
import jax.numpy as jnp
from jax import lax
from jax.experimental import pallas as pl
from jax.experimental.pallas import tpu as pltpu
from jax.experimental.pallas import tpu_sc as plsc

N_NODES = 10000
N_EDGES = 320000
D_FEAT = 128
N_HIDDEN = 128
N_CLASSES = 64

NC = 2
NS = 16
L = 16
NW = NC * NS

N_PAD = 10240
EDGE_K = 80

_MESH = dict(core_axis_name="c", subcore_axis_name="s")


def _newton_rsqrt(d):
    i = lax.bitcast_convert_type(d, jnp.int32)
    i = jnp.int32(0x5F3759DF) - lax.shift_right_logical(i, 1)
    y = lax.bitcast_convert_type(i, jnp.float32)
    for _ in range(3):
        y = y * (1.5 - 0.5 * d * y * y)
    return y


EPT_DEG = N_EDGES // NS
NODES_PER_TILE = N_PAD // NS


@functools.partial(
    pl.kernel,
    out_type=[
        jax.ShapeDtypeStruct((N_PAD,), jnp.float32),
        jax.ShapeDtypeStruct((N_PAD,), jnp.float32),
    ],
    mesh=plsc.VectorSubcoreMesh(**_MESH),
    compiler_params=pltpu.CompilerParams(
        needs_layout_passes=False, use_tc_tiling_on_sc=False
    ),
    scratch_types=[
        pltpu.VMEM((EPT_DEG,), jnp.int32),
        pltpu.VMEM((N_PAD,), jnp.float32),
        pltpu.VMEM((NS, NODES_PER_TILE), jnp.float32),
        pltpu.VMEM_SHARED((NS, N_PAD), jnp.float32),
    ],
)
def _sc_norms(adj_hbm, nsrc_hbm, ndst_hbm, idx_v, hist_v, stripe_v, parts_sh):
    c = lax.axis_index("c")
    s = lax.axis_index("s")

    pltpu.sync_copy(adj_hbm.at[c, pl.ds(s * EPT_DEG, EPT_DEG)], idx_v)

    zeros = jnp.zeros((L,), jnp.float32)

    def _zero(j, _):
        hist_v[pl.ds(j * L, L)] = zeros
        return 0

    lax.fori_loop(0, N_PAD // L, _zero, 0)

    ones = jnp.ones((L,), jnp.float32)

    def _count(i, _):
        idx = idx_v[pl.ds(i * L, L)]
        plsc.addupdate_scatter(hist_v, [idx], ones)
        return 0

    lax.fori_loop(0, EPT_DEG // L, _count, 0)

    pltpu.sync_copy(hist_v, parts_sh.at[s])
    plsc.subcore_barrier()
    base = s * NODES_PER_TILE
    pltpu.sync_copy(parts_sh.at[:, pl.ds(base, NODES_PER_TILE)], stripe_v)

    def _reduce(j, _):
        deg = jnp.zeros((L,), jnp.float32)
        for p in range(NS):
            deg = deg + stripe_v[p, pl.ds(j * L, L)]
        norm = jnp.where(deg > 0.0, _newton_rsqrt(jnp.maximum(deg, 1.0)), 0.0)
        hist_v[pl.ds(j * L, L)] = norm
        return 0

    lax.fori_loop(0, NODES_PER_TILE // L, _reduce, 0)

    @pl.when(c == 0)
    def _():
        pltpu.sync_copy(
            hist_v.at[pl.ds(0, NODES_PER_TILE)],
            nsrc_hbm.at[pl.ds(base, NODES_PER_TILE)],
        )

    @pl.when(c == 1)
    def _():
        pltpu.sync_copy(
            hist_v.at[pl.ds(0, NODES_PER_TILE)],
            ndst_hbm.at[pl.ds(base, NODES_PER_TILE)],
        )


EPT_AGG = N_EDGES // NW


AGG_K = 128
N_FULL = EPT_AGG // AGG_K
TAIL = EPT_AGG - N_FULL * AGG_K


def _make_sc_agg(feat):
    rows_per_tile = N_PAD // NS

    @functools.partial(
        pl.kernel,
        out_type=jax.ShapeDtypeStruct((NC, N_PAD, feat), jnp.float32),
        mesh=plsc.VectorSubcoreMesh(**_MESH),
        compiler_params=pltpu.CompilerParams(use_tc_tiling_on_sc=False),
        scratch_types=[
            pltpu.VMEM((EPT_AGG,), jnp.int32),
            [pltpu.VMEM((AGG_K,), jnp.int32)] * 2,
            pltpu.VMEM((TAIL,), jnp.int32),
            [pltpu.VMEM((AGG_K, feat), jnp.float32)] * 2,
            pltpu.VMEM_SHARED((N_PAD, feat), jnp.float32),
            [pltpu.SemaphoreType.DMA] * 2,
            [pltpu.SemaphoreType.DMA] * 2,
        ],
    )
    def _sc_agg(h_hbm, adj_hbm, out_hbm, src_v, didx, didx_t, rows,
                acc_sh, gsem, dsem):
        c = lax.axis_index("c")
        s = lax.axis_index("s")
        wid = s * NC + c
        base = wid * EPT_AGG

        pltpu.sync_copy(adj_hbm.at[0, pl.ds(base, EPT_AGG)], src_v)

        zeros = jnp.zeros((L,), jnp.float32)

        def _zrow(i, _):
            for t in range(feat // L):
                rows[0][i, pl.ds(t * L, L)] = zeros
            return 0

        lax.fori_loop(0, AGG_K, _zrow, 0)
        for t in range(rows_per_tile // AGG_K):
            pltpu.sync_copy(
                rows[0], acc_sh.at[pl.ds(s * rows_per_tile + t * AGG_K, AGG_K)]
            )
        plsc.subcore_barrier()

        def _gather(ci, b):
            pltpu.async_copy(
                h_hbm.at[src_v.at[pl.ds(ci * AGG_K, AGG_K)]], rows[b], gsem[b]
            )
            pltpu.async_copy(
                adj_hbm.at[1, pl.ds(base + ci * AGG_K, AGG_K)], didx[b], dsem[b]
            )

        def _wait_g(b):
            pltpu.make_async_copy(h_hbm.at[pl.ds(0, AGG_K)], rows[b], gsem[b]).wait()
            pltpu.make_async_copy(
                adj_hbm.at[1, pl.ds(base, AGG_K)], didx[b], dsem[b]
            ).wait()

        _gather(0, 0)

        def _body(i, _):
            c0 = 2 * i
            _gather(c0 + 1, 1)
            _wait_g(0)
            pltpu.sync_copy(rows[0], acc_sh.at[didx[0]], add=True)

            @pl.when(c0 + 2 < N_FULL)
            def _():
                _gather(c0 + 2, 0)

            _wait_g(1)
            pltpu.sync_copy(rows[1], acc_sh.at[didx[1]], add=True)
            return 0

        lax.fori_loop(0, N_FULL // 2, _body, 0)

        toff = base + N_FULL * AGG_K
        pltpu.sync_copy(adj_hbm.at[1, pl.ds(toff, TAIL)], didx_t)
        pltpu.async_copy(
            h_hbm.at[src_v.at[pl.ds(N_FULL * AGG_K, TAIL)]],
            rows[0].at[pl.ds(0, TAIL)],
            gsem[0],
        )
        pltpu.make_async_copy(
            h_hbm.at[pl.ds(0, TAIL)], rows[0].at[pl.ds(0, TAIL)], gsem[0]
        ).wait()
        pltpu.sync_copy(rows[0].at[pl.ds(0, TAIL)], acc_sh.at[didx_t], add=True)

        plsc.subcore_barrier()
        sl = pl.ds(s * rows_per_tile, rows_per_tile)
        pltpu.sync_copy(acc_sh.at[sl], out_hbm.at[c, sl])

    return _sc_agg


_sc_agg_h = _make_sc_agg(N_HIDDEN)
_sc_agg_c = _make_sc_agg(N_CLASSES)


BM = 512
GRID = N_PAD // BM


def _tc_mm1_body(x_ref, w_ref, o_ref):
    o_ref[...] = jnp.dot(x_ref[...], w_ref[...], preferred_element_type=jnp.float32)


def _tc_scale_body(y_ref, ns_ref, o_ref):
    ns = ns_ref[...]
    o_ref[...] = jnp.where(ns > 0.0, y_ref[...] * ns, 0.0)


def _tc_l2_body(p_ref, nd_ref, ns_ref, b_ref, w_ref, o_ref):
    t = (p_ref[0] + p_ref[1]) * nd_ref[...] + b_ref[...]
    t = jnp.maximum(t, 0.0) * ns_ref[...]
    o_ref[...] = jnp.dot(t, w_ref[...], preferred_element_type=jnp.float32)


def _tc_out_body(q_ref, nd_ref, b_ref, o_ref):
    z = (q_ref[0] + q_ref[1]) * nd_ref[...] + b_ref[...]
    m = jnp.max(z, axis=1, keepdims=True)
    e = jnp.exp(z - m)
    o_ref[...] = z - m - jnp.log(jnp.sum(e, axis=1, keepdims=True))


def _row_spec(cols):
    return pl.BlockSpec((BM, cols), lambda i: (i, 0))


def _full_spec(rows, cols):
    return pl.BlockSpec((rows, cols), lambda i: (0, 0))


def _pair_spec(cols):
    return pl.BlockSpec((2, BM, cols), lambda i: (0, i, 0))


def kernel(features, adj_metrix, W1, b1, W2, b2):
    y = pl.pallas_call(
        _tc_mm1_body,
        grid=(GRID,),
        in_specs=[_row_spec(D_FEAT), _full_spec(D_FEAT, N_HIDDEN)],
        out_specs=_row_spec(N_HIDDEN),
        out_shape=jax.ShapeDtypeStruct((N_PAD, N_HIDDEN), jnp.float32),
    )(features, W1)

    nsrc_flat, ndst_flat = _sc_norms(adj_metrix)
    nsrc = nsrc_flat.reshape(N_PAD, 1)
    ndst = ndst_flat.reshape(N_PAD, 1)

    h0 = pl.pallas_call(
        _tc_scale_body,
        grid=(GRID,),
        in_specs=[_row_spec(N_HIDDEN), _row_spec(1)],
        out_specs=_row_spec(N_HIDDEN),
        out_shape=jax.ShapeDtypeStruct((N_PAD, N_HIDDEN), jnp.float32),
    )(y, nsrc)

    p = _sc_agg_h(h0, adj_metrix)

    h1 = pl.pallas_call(
        _tc_l2_body,
        grid=(GRID,),
        in_specs=[
            _pair_spec(N_HIDDEN),
            _row_spec(1),
            _row_spec(1),
            _full_spec(1, N_HIDDEN),
            _full_spec(N_HIDDEN, N_CLASSES),
        ],
        out_specs=_row_spec(N_CLASSES),
        out_shape=jax.ShapeDtypeStruct((N_PAD, N_CLASSES), jnp.float32),
    )(p, ndst, nsrc, b1.reshape(1, N_HIDDEN), W2)

    q = _sc_agg_c(h1, adj_metrix)

    out = pl.pallas_call(
        _tc_out_body,
        grid=(GRID,),
        in_specs=[
            _pair_spec(N_CLASSES),
            _row_spec(1),
            _full_spec(1, N_CLASSES),
        ],
        out_specs=_row_spec(N_CLASSES),
        out_shape=jax.ShapeDtypeStruct((N_NODES, N_CLASSES), jnp.float32),
    )(q, ndst, b2.reshape(1, N_CLASSES))

    return out
